# baseline (speedup 1.0000x reference)
import functools

import jax
import jax.numpy as jnp
from jax import lax
from jax.experimental import pallas as pl
from jax.experimental.pallas import tpu as pltpu
from jax.experimental.pallas import tpu_sc as plsc

B, H, W, C = 4, 512, 512, 3
N = B * H * W
NC, NS, L = 2, 16, 16
NW = NC * NS
PW = N // NW
T = 4096
NT = PW // T


def _body(texc_hbm, mskw_hbm, cond_hbm, out_hbm, tex_v, msk_v, out_v, cond_v):
    cid = lax.axis_index("c")
    sid = lax.axis_index("s")
    wid = sid * NC + cid
    base = wid * PW
    img = wid // (NW // B)

    pltpu.sync_copy(cond_hbm, cond_v)
    sig = []
    for k in range(2 * C):
        idx = jnp.full((L,), img * (2 * C) + k, jnp.int32)
        v = plsc.load_gather(cond_v, [idx])
        sig.append(1.0 / (1.0 + jnp.exp(-v)))

    lanes = lax.broadcasted_iota(jnp.int32, (L,), 0)
    zero = jnp.zeros((L,), jnp.float32)

    for t in range(NT):
        po = base + t * T
        fo = pl.multiple_of(po * 3, 256)
        mo = pl.multiple_of(po // 4, 256)
        pltpu.sync_copy(texc_hbm.at[pl.ds(fo, 3 * T)], tex_v)
        pltpu.sync_copy(mskw_hbm.at[pl.ds(mo, T // 4)], msk_v)

        def inner(i):
            p = i * L + lanes
            p3 = p * 3
            z = plsc.load_gather(tex_v, [p3 + 2])
            w = plsc.load_gather(msk_v, [p >> 2])
            bit = (w >> ((p & 3) * 8)) & 1
            act = bit == 1
            m = z > -1.0
            for ch in range(C):
                val = jnp.where(m, sig[ch], sig[C + ch])
                val = jnp.where(act, val, zero)
                plsc.store_scatter(out_v, [p3 + ch], val)

        plsc.parallel_loop(0, T // L, 1, unroll=4)(inner)
        pltpu.sync_copy(out_v, out_hbm.at[pl.ds(fo, 3 * T)])


@jax.jit
def _run(texc_flat, mskw, cond_pad):
    mesh = plsc.VectorSubcoreMesh(core_axis_name="c", subcore_axis_name="s")
    return pl.kernel(
        _body,
        out_type=jax.ShapeDtypeStruct((N * C,), jnp.float32),
        mesh=mesh,
        scratch_types=[
            pltpu.VMEM((3 * T,), jnp.float32),
            pltpu.VMEM((T // 4,), jnp.int32),
            pltpu.VMEM((3 * T,), jnp.float32),
            pltpu.VMEM((32,), jnp.float32),
        ],
        compiler_params=pltpu.CompilerParams(needs_layout_passes=False),
    )(texc_flat, mskw, cond_pad)


def kernel(texc, view_dir, cond, msk):
    del view_dir
    texc_flat = texc.reshape(-1)
    mskw = lax.bitcast_convert_type(
        msk.reshape(-1, 4).astype(jnp.uint8), jnp.int32)
    cond_pad = jnp.concatenate(
        [cond.reshape(-1), jnp.zeros((8,), jnp.float32)])
    out = _run(texc_flat, mskw, cond_pad)
    return out.reshape(B, H, W, C)


# speedup vs baseline: 178.7293x; 178.7293x over previous
import functools

import jax
import jax.numpy as jnp
from jax.experimental import pallas as pl
from jax.experimental.pallas import tpu as pltpu

B, H, W, C = 4, 512, 512, 3
R = 512


def _body(z_ref, msk_ref, cond_ref, out_ref):
    z = z_ref[0, 0]
    mk = msk_ref[0] != 0
    cnd = cond_ref[0]
    sg = 1.0 / (1.0 + jnp.exp(-cnd))
    s_a = sg[0:C, :][:, None, :]
    s_b = sg[C:2 * C, :][:, None, :]
    m3 = (z > -1.0)[None, :, :]
    mk3 = mk[None, :, :]
    val = jnp.where(m3, s_a, s_b)
    val = jnp.where(mk3, val, 0.0)
    out_ref[0] = val


@jax.jit
def _run(texc_p, msk, conde):
    grid = (B, H // R)
    return pl.pallas_call(
        _body,
        grid=grid,
        in_specs=[
            pl.BlockSpec((1, 1, R, W), lambda i, j: (i, 2, j, 0)),
            pl.BlockSpec((1, R, W), lambda i, j: (i, j, 0)),
            pl.BlockSpec((1, 2 * C, W), lambda i, j: (i, 0, 0)),
        ],
        out_specs=pl.BlockSpec((1, C, R, W), lambda i, j: (i, 0, j, 0)),
        out_shape=jax.ShapeDtypeStruct((B, C, H, W), jnp.float32),
        compiler_params=pltpu.CompilerParams(
            dimension_semantics=("parallel", "parallel")),
    )(texc_p, msk, conde)


def kernel(texc, view_dir, cond, msk):
    del view_dir
    texc_p = jnp.transpose(texc, (0, 3, 1, 2))
    conde = jnp.broadcast_to(cond[:, :, None], (B, 2 * C, W))
    out_p = _run(texc_p, msk.astype(jnp.int8), conde)
    return jnp.transpose(out_p, (0, 2, 3, 1))
